# baseline (speedup 1.0000x reference)
import functools

import jax
import jax.numpy as jnp
from jax.experimental import pallas as pl
from jax.experimental.pallas import tpu as pltpu

_LANE = 128
_TILE_B = 2048
_WCHUNKS = 2
_VMEM_LIMIT = 64 * 1024 * 1024


def _matmul_bias(x, w, b):
    logits = jax.lax.dot_general(
        x, w, dimension_numbers=(((1,), (1,)), ((), ())),
        preferred_element_type=jnp.float32)
    return logits + b


def _chunked_body(x_ref, w_ref, b_ref, o_ref):
    j = pl.program_id(2)
    t = x_ref.shape[0]
    o_ref[pl.ds(j * t, t), :] = _matmul_bias(
        x_ref[...], w_ref[...], b_ref[...]).astype(o_ref.dtype)


def _flat_body(x_ref, w_ref, b_ref, o_ref):
    n = o_ref.shape[-1]
    o_ref[...] = _matmul_bias(
        x_ref[...], w_ref[...], b_ref[...])[:, :n].astype(o_ref.dtype)


def _pick_tile(B):
    if B <= _TILE_B:
        return B
    t = _TILE_B
    while B % t and t > 8:
        t //= 2
    return t


@jax.jit
def kernel(pooled_output, weight, bias):
    B, H = pooled_output.shape
    L = weight.shape[0]

    Lp = pl.cdiv(L, _LANE) * _LANE
    w_p = weight
    bias_p = bias
    if Lp != L:
        w_p = jnp.pad(weight, ((0, Lp - L), (0, 0)))
        bias_p = jnp.pad(bias, (0, Lp - L))
    b2 = bias_p.reshape(1, Lp)

    tile_b = _pick_tile(B)
    cost = pl.CostEstimate(
        flops=2 * B * H * Lp,
        transcendentals=0,
        bytes_accessed=B * H * 4 + Lp * H * 4 + B * L * 4)
    out_shape = jax.ShapeDtypeStruct((B, L), pooled_output.dtype)

    steps_per_core = B // (2 * _WCHUNKS * tile_b)
    if (L == Lp and steps_per_core >= 1
            and 2 * _WCHUNKS * steps_per_core * tile_b == B):
        chunk_rows = steps_per_core * tile_b
        return pl.pallas_call(
            _chunked_body,
            grid=(2, _WCHUNKS, steps_per_core),
            in_specs=[
                pl.BlockSpec(
                    (tile_b, H),
                    lambda c, k, j, s=steps_per_core, w=_WCHUNKS:
                        ((c * w + k) * s + j, 0)),
                pl.BlockSpec((Lp, H), lambda c, k, j: (0, 0)),
                pl.BlockSpec((1, Lp), lambda c, k, j: (0, 0)),
            ],
            out_specs=pl.BlockSpec(
                (chunk_rows, L), lambda c, k, j, w=_WCHUNKS: (c * w + k, 0)),
            out_shape=out_shape,
            compiler_params=pltpu.CompilerParams(
                dimension_semantics=("parallel", "arbitrary", "arbitrary"),
                vmem_limit_bytes=_VMEM_LIMIT),
            cost_estimate=cost,
        )(pooled_output, w_p, b2)

    return pl.pallas_call(
        _flat_body,
        grid=(pl.cdiv(B, tile_b),),
        in_specs=[
            pl.BlockSpec((tile_b, H), lambda i: (i, 0)),
            pl.BlockSpec((Lp, H), lambda i: (0, 0)),
            pl.BlockSpec((1, Lp), lambda i: (0, 0)),
        ],
        out_specs=pl.BlockSpec((tile_b, L), lambda i: (i, 0)),
        out_shape=out_shape,
        compiler_params=pltpu.CompilerParams(
            dimension_semantics=("parallel",),
            vmem_limit_bytes=_VMEM_LIMIT),
        cost_estimate=cost,
    )(pooled_output, w_p, b2)
